# baseline (speedup 1.0000x reference)
import functools

import jax
import jax.numpy as jnp
from jax import lax
from jax.experimental import pallas as pl
from jax.experimental.pallas import tpu as pltpu
from jax.experimental.pallas import tpu_sc as plsc

R = 128
C = 32768
L = 16
NC = 2
NS = 16
NW = NC * NS
RPW = R // NW
ZWORDS = 4096
NZ = C // ZWORDS
BR = 8
SEG = 128

_mesh = plsc.VectorSubcoreMesh(core_axis_name="c", subcore_axis_name="s")


@functools.partial(
    pl.kernel,
    mesh=_mesh,
    out_type=jax.ShapeDtypeStruct((R, C), jnp.float32),
    scratch_types=[
        pltpu.VMEM((ZWORDS,), jnp.float32),
        pltpu.SemaphoreType.DMA,
    ],
)
def _sc_zeros(out_hbm, zbuf, sem_w):
    wid = lax.axis_index("s") * NC + lax.axis_index("c")
    row0 = wid * RPW

    zero16 = jnp.zeros((L,), jnp.float32)

    def _zfill(i, _):
        zbuf[pl.ds(pl.multiple_of(i * L, L), L)] = zero16
        return 0
    lax.fori_loop(0, ZWORDS // L, _zfill, 0)

    zw = []
    for r in range(RPW):
        for z in range(NZ):
            dst = out_hbm.at[row0 + r, pl.ds(pl.multiple_of(z * ZWORDS, L),
                                             ZWORDS)]
            zw.append(pltpu.async_copy(zbuf, dst, sem_w))
    for h in zw:
        h.wait()


def _amax_body(x_ref, v_ref, l_ref, s_ref):
    xb = x_ref[...]
    v = jnp.max(xb, axis=1)
    c = jnp.argmax(xb, axis=1).astype(jnp.int32)
    v_ref[...] = v.reshape(1, 1, BR)
    l_ref[...] = (c & (SEG - 1)).reshape(1, 1, BR)
    s_ref[...] = (c >> 7).reshape(1, 1, BR)


def _poke_body(segs_ref, vals_ref, lanes_ref, zeros_ref, out_ref,
               pbuf, sem):
    lane_iota = lax.broadcasted_iota(jnp.int32, (R, SEG), 1)
    pbuf[...] = jnp.where(lane_iota == lanes_ref[...], vals_ref[...],
                          jnp.float32(0.0))

    handles = []
    for r in range(R):
        seg = segs_ref[r]
        dst = out_ref.at[pl.ds(r, 1), pl.ds(seg * SEG, SEG)]
        cp = pltpu.make_async_copy(pbuf.at[pl.ds(r, 1), :], dst, sem)
        cp.start()
        handles.append(cp)
        if len(handles) == 32:
            for h in handles:
                h.wait()
            handles = []
    for h in handles:
        h.wait()


def kernel(x):
    zeros = _sc_zeros()
    v3, l3, s3 = pl.pallas_call(
        _amax_body,
        grid=(R // BR,),
        in_specs=[pl.BlockSpec((BR, C), lambda i: (i, 0))],
        out_specs=[pl.BlockSpec((1, 1, BR), lambda i: (i, 0, 0)),
                   pl.BlockSpec((1, 1, BR), lambda i: (i, 0, 0)),
                   pl.BlockSpec((1, 1, BR), lambda i: (i, 0, 0))],
        out_shape=[jax.ShapeDtypeStruct((R // BR, 1, BR), jnp.float32),
                   jax.ShapeDtypeStruct((R // BR, 1, BR), jnp.int32),
                   jax.ShapeDtypeStruct((R // BR, 1, BR), jnp.int32)],
    )(x)
    out = pl.pallas_call(
        _poke_body,
        in_specs=[
            pl.BlockSpec(memory_space=pltpu.SMEM),
            pl.BlockSpec(memory_space=pltpu.VMEM),
            pl.BlockSpec(memory_space=pltpu.VMEM),
            pl.BlockSpec(memory_space=pl.ANY),
        ],
        out_specs=pl.BlockSpec(memory_space=pl.ANY),
        out_shape=jax.ShapeDtypeStruct((R, C), jnp.float32),
        scratch_shapes=[pltpu.VMEM((R, SEG), jnp.float32),
                        pltpu.SemaphoreType.DMA],
        input_output_aliases={3: 0},
    )(s3.reshape(R), v3.reshape(R, 1), l3.reshape(R, 1), zeros)
    return out


# speedup vs baseline: 1.1160x; 1.1160x over previous
import functools

import jax
import jax.numpy as jnp
from jax import lax
from jax.experimental import pallas as pl
from jax.experimental.pallas import tpu as pltpu
from jax.experimental.pallas import tpu_sc as plsc

R = 128
C = 32768
L = 16
NC = 2
NS = 16
NW = NC * NS
RPW = R // NW
ZWORDS = 4096
NZ = C // ZWORDS
BR = 8
SEG = 128

_mesh = plsc.VectorSubcoreMesh(core_axis_name="c", subcore_axis_name="s")


@functools.partial(
    pl.kernel,
    mesh=_mesh,
    out_type=jax.ShapeDtypeStruct((R, C), jnp.float32),
    scratch_types=[
        pltpu.VMEM((ZWORDS,), jnp.float32),
        pltpu.SemaphoreType.DMA,
    ],
)
def _sc_zeros(out_hbm, zbuf, sem_w):
    wid = lax.axis_index("s") * NC + lax.axis_index("c")
    row0 = wid * RPW

    zero16 = jnp.zeros((L,), jnp.float32)

    def _zfill(i, _):
        zbuf[pl.ds(pl.multiple_of(i * L, L), L)] = zero16
        return 0
    lax.fori_loop(0, ZWORDS // L, _zfill, 0)

    zw = []
    for r in range(RPW):
        for z in range(NZ):
            dst = out_hbm.at[row0 + r, pl.ds(pl.multiple_of(z * ZWORDS, L),
                                             ZWORDS)]
            zw.append(pltpu.async_copy(zbuf, dst, sem_w))
    for h in zw:
        h.wait()


def _amax_body(x_ref, v_ref, l_ref, s_ref):
    xb = x_ref[...]
    v = jnp.max(xb, axis=1, keepdims=True)
    iota = lax.broadcasted_iota(jnp.int32, (BR, C), 1)
    c = jnp.min(jnp.where(xb == v, iota, jnp.int32(C)), axis=1)
    c = c.reshape(BR, 1)
    v_ref[...] = v
    l_ref[...] = c & (SEG - 1)
    s_ref[...] = c >> 7


def _poke_body(segs_ref, vals_ref, lanes_ref, zeros_ref, out_ref,
               pbuf, sem):
    lane_iota = lax.broadcasted_iota(jnp.int32, (R, SEG), 1)
    pbuf[...] = jnp.where(lane_iota == lanes_ref[...], vals_ref[...],
                          jnp.float32(0.0))

    handles = []
    for r in range(R):
        seg = segs_ref[r, 0]
        dst = out_ref.at[pl.ds(r, 1), pl.ds(seg * SEG, SEG)]
        cp = pltpu.make_async_copy(pbuf.at[pl.ds(r, 1), :], dst, sem)
        cp.start()
        handles.append(cp)
        if len(handles) == 32:
            for h in handles:
                h.wait()
            handles = []
    for h in handles:
        h.wait()


def kernel(x):
    zeros = _sc_zeros()
    v3, l3, s3 = pl.pallas_call(
        _amax_body,
        grid=(R // BR,),
        in_specs=[pl.BlockSpec((BR, C), lambda i: (i, 0))],
        out_specs=[pl.BlockSpec((BR, 1), lambda i: (i, 0)),
                   pl.BlockSpec((BR, 1), lambda i: (i, 0)),
                   pl.BlockSpec((BR, 1), lambda i: (i, 0))],
        out_shape=[jax.ShapeDtypeStruct((R, 1), jnp.float32),
                   jax.ShapeDtypeStruct((R, 1), jnp.int32),
                   jax.ShapeDtypeStruct((R, 1), jnp.int32)],
    )(x)
    out = pl.pallas_call(
        _poke_body,
        in_specs=[
            pl.BlockSpec(memory_space=pltpu.SMEM),
            pl.BlockSpec(memory_space=pltpu.VMEM),
            pl.BlockSpec(memory_space=pltpu.VMEM),
            pl.BlockSpec(memory_space=pl.ANY),
        ],
        out_specs=pl.BlockSpec(memory_space=pl.ANY),
        out_shape=jax.ShapeDtypeStruct((R, C), jnp.float32),
        scratch_shapes=[pltpu.VMEM((R, SEG), jnp.float32),
                        pltpu.SemaphoreType.DMA],
        input_output_aliases={3: 0},
    )(s3, v3, l3, zeros)
    return out


# speedup vs baseline: 1.2009x; 1.0761x over previous
import functools

import jax
import jax.numpy as jnp
from jax import lax
from jax.experimental import pallas as pl
from jax.experimental.pallas import tpu as pltpu
from jax.experimental.pallas import tpu_sc as plsc

R = 128
C = 32768
L = 16
NC = 2
NS = 16
NW = NC * NS
RPW = R // NW
UNROLL = 8
NJ = C // (L * UNROLL)
BR = 8
SEG = 128

_mesh = plsc.VectorSubcoreMesh(core_axis_name="c", subcore_axis_name="s")


@functools.partial(
    pl.kernel,
    mesh=_mesh,
    out_type=[
        jax.ShapeDtypeStruct((R, SEG), jnp.float32),
        jax.ShapeDtypeStruct((R, SEG), jnp.int32),
    ],
    scratch_types=[
        pltpu.VMEM((C,), jnp.float32),
        pltpu.VMEM((C,), jnp.float32),
        pltpu.VMEM((RPW * SEG,), jnp.float32),
        pltpu.VMEM((RPW * SEG,), jnp.int32),
        pltpu.SemaphoreType.DMA,
        pltpu.SemaphoreType.DMA,
        pltpu.SemaphoreType.DMA,
    ],
)
def _sc_amax(x_hbm, poke_hbm, seg_hbm, buf0, buf1, pbuf, sbuf,
             sem_r0, sem_r1, sem_w):
    wid = lax.axis_index("s") * NC + lax.axis_index("c")
    row0 = wid * RPW

    iota = lax.iota(jnp.int32, L)
    zero16 = jnp.zeros((L,), jnp.float32)

    bufs = (buf0, buf1)
    sems = (sem_r0, sem_r1)
    rd = [None] * RPW
    rd[0] = pltpu.async_copy(x_hbm.at[row0], buf0, sem_r0)
    rd[1] = pltpu.async_copy(x_hbm.at[row0 + 1], buf1, sem_r1)

    wr = []
    for r in range(RPW):
        buf = bufs[r % 2]
        rd[r].wait()

        neg_inf = jnp.full((L,), -jnp.inf, jnp.float32)
        bv0 = tuple(neg_inf for _ in range(UNROLL))
        bj0 = tuple(jnp.zeros((L,), jnp.int32) for _ in range(UNROLL))

        def _step(j, carry):
            bvs, bjs = carry
            base = pl.multiple_of(j * (L * UNROLL), L * UNROLL)
            jb = jnp.full((L,), j, jnp.int32)
            nbvs, nbjs = [], []
            for k in range(UNROLL):
                v = buf[pl.ds(base + k * L, L)]
                gt = v > bvs[k]
                nbvs.append(jnp.maximum(bvs[k], v))
                nbjs.append(jnp.where(gt, jb, bjs[k]))
            return tuple(nbvs), tuple(nbjs)

        bvs, bjs = lax.fori_loop(0, NJ, _step, (bv0, bj0))

        mval, mcol = None, None
        for k in range(UNROLL):
            col = bjs[k] * (L * UNROLL) + (k * L + iota)
            if mval is None:
                mval, mcol = bvs[k], col
            else:
                take = (bvs[k] > mval) | ((bvs[k] == mval) & (col < mcol))
                mval = jnp.where(take, bvs[k], mval)
                mcol = jnp.where(take, col, mcol)

        rmax = mval[0]
        rcol = mcol[0]
        for l in range(1, L):
            v = mval[l]
            c = mcol[l]
            take = (v > rmax) | ((v == rmax) & (c < rcol))
            rmax = jnp.where(take, v, rmax)
            rcol = jnp.where(take, c, rcol)

        seg = lax.shift_right_logical(rcol, 7)
        lane = rcol - seg * SEG

        for t in range(SEG // L):
            piece = jnp.where(iota + (t * L) == lane,
                              jnp.full((L,), rmax), zero16)
            pbuf[pl.ds(r * SEG + t * L, L)] = piece
        for t in range(SEG // L):
            sbuf[pl.ds(r * SEG + t * L, L)] = jnp.full((L,), seg, jnp.int32)

        wr.append(pltpu.async_copy(pbuf.at[pl.ds(r * SEG, SEG)],
                                   poke_hbm.at[row0 + r], sem_w))
        wr.append(pltpu.async_copy(sbuf.at[pl.ds(r * SEG, SEG)],
                                   seg_hbm.at[row0 + r], sem_w))

        if r + 2 < RPW:
            rd[r + 2] = pltpu.async_copy(x_hbm.at[row0 + r + 2],
                                         bufs[r % 2], sems[r % 2])

    for h in wr:
        h.wait()


def _zeros_body(out_ref):
    out_ref[...] = jnp.zeros((BR, C), jnp.float32)


def _poke_body(segs_ref, pokes_ref, zeros_ref, out_ref, sem):
    handles = []
    for r in range(R):
        seg = segs_ref[r, 0]
        dst = out_ref.at[pl.ds(r, 1), pl.ds(seg * SEG, SEG)]
        cp = pltpu.make_async_copy(pokes_ref.at[pl.ds(r, 1), :], dst, sem)
        cp.start()
        handles.append(cp)
        if len(handles) == 32:
            for h in handles:
                h.wait()
            handles = []
    for h in handles:
        h.wait()


def kernel(x):
    pokes, segs = _sc_amax(x)
    zeros = pl.pallas_call(
        _zeros_body,
        grid=(R // BR,),
        out_specs=pl.BlockSpec((BR, C), lambda i: (i, 0)),
        out_shape=jax.ShapeDtypeStruct((R, C), jnp.float32),
    )()
    out = pl.pallas_call(
        _poke_body,
        in_specs=[
            pl.BlockSpec(memory_space=pltpu.SMEM),
            pl.BlockSpec(memory_space=pltpu.VMEM),
            pl.BlockSpec(memory_space=pl.ANY),
        ],
        out_specs=pl.BlockSpec(memory_space=pl.ANY),
        out_shape=jax.ShapeDtypeStruct((R, C), jnp.float32),
        scratch_shapes=[pltpu.SemaphoreType.DMA],
        input_output_aliases={2: 0},
    )(segs, pokes, zeros)
    return out


# speedup vs baseline: 1.2605x; 1.0496x over previous
import functools

import jax
import jax.numpy as jnp
from jax import lax
from jax.experimental import pallas as pl
from jax.experimental.pallas import tpu as pltpu
from jax.experimental.pallas import tpu_sc as plsc

R = 128
C = 32768
L = 16
NC = 2
NS = 16
NW = NC * NS
RPW = R // NW
UNROLL = 8
NJ = C // (L * UNROLL)
BR = 8
SEG = 128

_mesh = plsc.VectorSubcoreMesh(core_axis_name="c", subcore_axis_name="s")


@functools.partial(
    pl.kernel,
    mesh=_mesh,
    out_type=[
        jax.ShapeDtypeStruct((R, SEG), jnp.float32),
        jax.ShapeDtypeStruct((R, SEG), jnp.int32),
    ],
    scratch_types=[
        pltpu.VMEM((C,), jnp.float32),
        pltpu.VMEM((C,), jnp.float32),
        pltpu.VMEM((RPW * SEG,), jnp.float32),
        pltpu.VMEM((RPW * SEG,), jnp.int32),
        pltpu.SemaphoreType.DMA,
        pltpu.SemaphoreType.DMA,
        pltpu.SemaphoreType.DMA,
    ],
)
def _sc_amax(x_hbm, poke_hbm, seg_hbm, buf0, buf1, pbuf, sbuf,
             sem_r0, sem_r1, sem_w):
    wid = lax.axis_index("s") * NC + lax.axis_index("c")
    row0 = wid * RPW

    iota = lax.iota(jnp.int32, L)
    zero16 = jnp.zeros((L,), jnp.float32)

    bufs = (buf0, buf1)
    sems = (sem_r0, sem_r1)
    rd = [None] * RPW
    rd[0] = pltpu.async_copy(x_hbm.at[row0], buf0, sem_r0)
    rd[1] = pltpu.async_copy(x_hbm.at[row0 + 1], buf1, sem_r1)

    wr = []
    for r in range(RPW):
        buf = bufs[r % 2]
        rd[r].wait()

        neg_inf = jnp.full((L,), -jnp.inf, jnp.float32)
        bv0 = tuple(neg_inf for _ in range(UNROLL))
        bj0 = tuple(jnp.zeros((L,), jnp.int32) for _ in range(UNROLL))

        def _step(j, carry):
            bvs, bjs = carry
            base = pl.multiple_of(j * (L * UNROLL), L * UNROLL)
            jb = jnp.full((L,), j, jnp.int32)
            nbvs, nbjs = [], []
            for k in range(UNROLL):
                v = buf[pl.ds(base + k * L, L)]
                gt = v > bvs[k]
                nbvs.append(jnp.maximum(bvs[k], v))
                nbjs.append(jnp.where(gt, jb, bjs[k]))
            return tuple(nbvs), tuple(nbjs)

        bvs, bjs = lax.fori_loop(0, NJ, _step, (bv0, bj0))

        mval, mcol = None, None
        for k in range(UNROLL):
            col = bjs[k] * (L * UNROLL) + (k * L + iota)
            if mval is None:
                mval, mcol = bvs[k], col
            else:
                take = (bvs[k] > mval) | ((bvs[k] == mval) & (col < mcol))
                mval = jnp.where(take, bvs[k], mval)
                mcol = jnp.where(take, col, mcol)

        rmax = mval[0]
        rcol = mcol[0]
        for l in range(1, L):
            v = mval[l]
            c = mcol[l]
            take = (v > rmax) | ((v == rmax) & (c < rcol))
            rmax = jnp.where(take, v, rmax)
            rcol = jnp.where(take, c, rcol)

        seg = lax.shift_right_logical(rcol, 7)
        lane = rcol - seg * SEG

        for t in range(SEG // L):
            piece = jnp.where(iota + (t * L) == lane,
                              jnp.full((L,), rmax), zero16)
            pbuf[pl.ds(r * SEG + t * L, L)] = piece
        for t in range(SEG // L):
            sbuf[pl.ds(r * SEG + t * L, L)] = jnp.full((L,), seg, jnp.int32)

        wr.append(pltpu.async_copy(pbuf.at[pl.ds(r * SEG, SEG)],
                                   poke_hbm.at[row0 + r], sem_w))
        wr.append(pltpu.async_copy(sbuf.at[pl.ds(r * SEG, SEG)],
                                   seg_hbm.at[row0 + r], sem_w))

        if r + 2 < RPW:
            rd[r + 2] = pltpu.async_copy(x_hbm.at[row0 + r + 2],
                                         bufs[r % 2], sems[r % 2])

    for h in wr:
        h.wait()


def _zeros_body(out_ref):
    out_ref[...] = jnp.zeros((BR, C), jnp.float32)


def _poke_body(segs_ref, pokes_ref, zeros_ref, out_ref, sem):
    handles = []
    for r in range(R):
        seg = segs_ref[r, 0]
        dst = out_ref.at[pl.ds(r, 1), pl.ds(seg * SEG, SEG)]
        cp = pltpu.make_async_copy(pokes_ref.at[pl.ds(r, 1), :], dst, sem)
        cp.start()
        handles.append(cp)
    for h in handles:
        h.wait()


def kernel(x):
    pokes, segs = _sc_amax(x)
    zeros = pl.pallas_call(
        _zeros_body,
        grid=(R // BR,),
        out_specs=pl.BlockSpec((BR, C), lambda i: (i, 0)),
        out_shape=jax.ShapeDtypeStruct((R, C), jnp.float32),
    )()
    out = pl.pallas_call(
        _poke_body,
        in_specs=[
            pl.BlockSpec(memory_space=pltpu.SMEM),
            pl.BlockSpec(memory_space=pltpu.VMEM),
            pl.BlockSpec(memory_space=pl.ANY),
        ],
        out_specs=pl.BlockSpec(memory_space=pl.ANY),
        out_shape=jax.ShapeDtypeStruct((R, C), jnp.float32),
        scratch_shapes=[pltpu.SemaphoreType.DMA],
        input_output_aliases={2: 0},
    )(segs, pokes, zeros)
    return out
